# baseline (speedup 1.0000x reference)
import jax
import jax.numpy as jnp
from jax import lax
from jax.experimental import pallas as pl
from jax.experimental.pallas import tpu as pltpu
from jax.experimental.pallas import tpu_sc as plsc

N_GRAM = 4
D_STATE = 16
N_SLOTS = 500000
MOMENTUM = 0.9
N_TOK = 4 * 4096
N_SUB = 16
TPW = N_TOK // N_SUB
CHUNK = 128
N_CH = TPW // CHUNK
L = 16


def _body(twT, stT, bank, counts, ro,
          sh_col, tok_v, addr_v, hcv, cntv, bkv, ssv, colv, gv,
          candv, ncv, zov, onev, alv, bev):
  wid = lax.axis_index("s")
  base = wid * TPW

  for j in range(N_GRAM):
    pltpu.sync_copy(twT.at[j, pl.ds(base, TPW)], tok_v.at[j])

  @pl.loop(0, TPW // L)
  def _hash(k):
    off = k * L
    h = jnp.full((L,), 2166136261, jnp.uint32)
    for j in range(N_GRAM):
      t = tok_v[j, pl.ds(off, L)].astype(jnp.uint32)
      h = (h ^ t) * jnp.uint32(16777619)
    a = (h % jnp.uint32(N_SLOTS)).astype(jnp.int32)
    addr_v[k // (CHUNK // L), pl.ds((k % (CHUNK // L)) * L, L)] = a
    zov[pl.ds(off, L)] = jnp.zeros((L,), jnp.float32)
    onev[pl.ds(off, L)] = jnp.ones((L,), jnp.float32)

  for c in range(N_CH):
    idx = addr_v.at[c]
    pltpu.sync_copy(counts.at[idx], cntv.at[pl.ds(c * CHUNK, CHUNK)])
    pltpu.sync_copy(bank.at[idx], bkv.at[pl.ds(c * CHUNK, CHUNK)])

  for c in range(N_CH):
    pltpu.sync_copy(zov.at[pl.ds(c * CHUNK, CHUNK)], sh_col.at[addr_v.at[c]])
  plsc.subcore_barrier()
  for c in range(N_CH):
    pltpu.sync_copy(onev.at[pl.ds(c * CHUNK, CHUNK)], sh_col.at[addr_v.at[c]],
                    add=True)
  plsc.subcore_barrier()
  for c in range(N_CH):
    pltpu.sync_copy(sh_col.at[addr_v.at[c]], hcv.at[pl.ds(c * CHUNK, CHUNK)])

  @pl.loop(0, D_STATE)
  def _col(d):
    plsc.subcore_barrier()
    for c in range(N_CH):
      pltpu.sync_copy(zov.at[pl.ds(c * CHUNK, CHUNK)],
                      sh_col.at[addr_v.at[c]])
    pltpu.sync_copy(stT.at[d, pl.ds(base, TPW)], colv)
    plsc.subcore_barrier()
    for c in range(N_CH):
      pltpu.sync_copy(colv.at[pl.ds(c * CHUNK, CHUNK)],
                      sh_col.at[addr_v.at[c]], add=True)
    plsc.subcore_barrier()
    for c in range(N_CH):
      pltpu.sync_copy(sh_col.at[addr_v.at[c]], gv.at[pl.ds(c * CHUNK, CHUNK)])

    @pl.loop(0, TPW // L)
    def _tr(k):
      v = gv[pl.ds(k * L, L)]
      flat = (k * L + lax.iota(jnp.int32, L)) * D_STATE + d
      plsc.store_scatter(ssv, [flat], v)

  @pl.loop(0, TPW // L)
  def _coef(k):
    off = k * L
    hc = hcv[pl.ds(off, L)]
    cnt = cntv[pl.ds(off, L)]
    alpha = jnp.where(cnt == 0, jnp.float32(0.0), jnp.float32(MOMENTUM))
    alv[pl.ds(off, L)] = alpha
    bev[pl.ds(off, L)] = (jnp.float32(1.0) - alpha) / hc
    ncv[pl.ds(off, L)] = cnt + hc.astype(jnp.int32)

  @pl.loop(0, TPW // L)
  def _cand(k):
    av = alv[pl.ds(k * L, L)]
    bv = bev[pl.ds(k * L, L)]
    for i in range(L):
      t = k * L + i
      srow = ssv[pl.ds(t * D_STATE, D_STATE)]
      candv[t] = av[i] * bkv[t] + bv[i] * srow

  plsc.subcore_barrier()
  for c in range(N_CH):
    idx = addr_v.at[c]
    pltpu.sync_copy(candv.at[pl.ds(c * CHUNK, CHUNK)], bank.at[idx])
    pltpu.sync_copy(ncv.at[pl.ds(c * CHUNK, CHUNK)], counts.at[idx])
  pltpu.sync_copy(candv, ro.at[pl.ds(base, TPW)])


def kernel(token_window, states, bank, counts):
  twT = token_window.reshape(N_TOK, N_GRAM).T
  stT = states.reshape(N_TOK, D_STATE).astype(jnp.float32).T

  mesh = plsc.VectorSubcoreMesh(
      core_axis_name="c", subcore_axis_name="s", num_cores=1)
  run = pl.kernel(
      _body,
      out_type=jax.ShapeDtypeStruct((N_TOK, D_STATE), jnp.float32),
      mesh=mesh,
      scratch_types=[
          pltpu.VMEM_SHARED((N_SLOTS,), jnp.float32),
          pltpu.VMEM((N_GRAM, TPW), jnp.int32),
          pltpu.VMEM((N_CH, CHUNK), jnp.int32),
          pltpu.VMEM((TPW,), jnp.float32),
          pltpu.VMEM((TPW,), jnp.int32),
          pltpu.VMEM((TPW, D_STATE), jnp.float32),
          pltpu.VMEM((TPW * D_STATE,), jnp.float32),
          pltpu.VMEM((TPW,), jnp.float32),
          pltpu.VMEM((TPW,), jnp.float32),
          pltpu.VMEM((TPW, D_STATE), jnp.float32),
          pltpu.VMEM((TPW,), jnp.int32),
          pltpu.VMEM((TPW,), jnp.float32),
          pltpu.VMEM((TPW,), jnp.float32),
          pltpu.VMEM((TPW,), jnp.float32),
          pltpu.VMEM((TPW,), jnp.float32),
      ],
      compiler_params=pltpu.CompilerParams(
          needs_layout_passes=False, use_tc_tiling_on_sc=False),
      name="token_memory_bank_sc",
  )

  bank_ref = jax.new_ref(bank)
  counts_ref = jax.new_ref(counts)
  ro = run(twT, stT, bank_ref, counts_ref)
  new_bank = bank_ref[...]
  new_counts = counts_ref[...]
  read_out = ro.reshape(token_window.shape[0], token_window.shape[1], D_STATE)
  return new_bank, new_counts, read_out
